# Initial kernel scaffold; baseline (speedup 1.0000x reference)
import jax
import jax.numpy as jnp
from jax.experimental import pallas as pl


def kernel(x, target):
    raise NotImplementedError("write your pallas kernel here")


# speedup vs baseline: 5.0951x; 5.0951x over previous
import functools

import jax
import jax.numpy as jnp
from jax import lax
from jax.experimental import pallas as pl
from jax.experimental.pallas import tpu as pltpu
from jax.experimental.pallas import tpu_sc as plsc

_V = 16384
_N = 4 * 2048
_PAD = 0
_NC = 2
_NS = 16
_NW = _NC * _NS
_RPW = _N // _NW
_CH = 128
_NCH = _RPW // _CH
_L = 16


def _make_loss_kernel():
    mesh = plsc.VectorSubcoreMesh(core_axis_name="c", subcore_axis_name="s")

    @functools.partial(
        pl.kernel,
        out_type=jax.ShapeDtypeStruct((_NC, _L), jnp.float32),
        mesh=mesh,
        scratch_types=[
            pltpu.VMEM((_RPW,), jnp.int32),
            pltpu.VMEM((_NCH, _CH), jnp.int32),
            pltpu.VMEM((_NCH, _CH), jnp.float32),
            pltpu.VMEM((_L,), jnp.float32),
            pltpu.VMEM((_NS * _L,), jnp.float32),
            pltpu.VMEM_SHARED((_NS * _L,), jnp.float32),
            pltpu.SemaphoreType.DMA,
        ],
    )
    def loss_kernel(x_hbm, t_hbm, out_hbm, t_v, idx_v, val_v, acc_v, all_v,
                    part_sh, sem):
        cid = lax.axis_index("c")
        sid = lax.axis_index("s")
        wid = sid * _NC + cid
        base = wid * _RPW

        pltpu.sync_copy(t_hbm.at[pl.ds(base, _RPW)], t_v)

        lanes = lax.iota(jnp.int32, _L)
        for c in range(_NCH):
            for i in range(_CH // _L):
                off = c * _CH + i * _L
                tv = t_v[pl.ds(off, _L)]
                idx_v[c, pl.ds(i * _L, _L)] = (base + off + lanes) * _V + tv

        copies = [
            pltpu.async_copy(x_hbm.at[idx_v.at[c]], val_v.at[c], sem)
            for c in range(_NCH)
        ]
        for cp in copies:
            cp.wait()

        acc = jnp.zeros((_L,), jnp.float32)
        for c in range(_NCH):
            for i in range(_CH // _L):
                off = c * _CH + i * _L
                tv = t_v[pl.ds(off, _L)]
                v = val_v[c, pl.ds(i * _L, _L)]
                acc = acc - jnp.where(tv != _PAD, v, 0.0)

        acc_v[...] = acc
        pltpu.sync_copy(acc_v, part_sh.at[pl.ds(sid * _L, _L)])
        plsc.subcore_barrier()

        @pl.when(sid == 0)
        def _():
            pltpu.sync_copy(part_sh, all_v)
            tot = jnp.zeros((_L,), jnp.float32)
            for s in range(_NS):
                tot = tot + all_v[pl.ds(s * _L, _L)]
            acc_v[...] = tot
            pltpu.sync_copy(acc_v, out_hbm.at[cid])

    return loss_kernel


_loss_kernel = _make_loss_kernel()


@jax.jit
def kernel(x, target):
    xf = x.reshape(-1)
    t = target.reshape(-1).astype(jnp.int32)
    partials = _loss_kernel(xf, t)
    return jnp.sum(partials)


# speedup vs baseline: 88.4656x; 17.3629x over previous
import functools

import jax
import jax.numpy as jnp
from jax import lax
from jax.experimental import pallas as pl
from jax.experimental.pallas import tpu as pltpu
from jax.experimental.pallas import tpu_sc as plsc

_V = 16384
_N = 4 * 2048
_PAD = 0
_NC = 2
_NS = 16
_NW = _NC * _NS
_RPW = _N // _NW
_CH = 128
_NCH = _RPW // _CH
_L = 16


def _make_loss_kernel():
    mesh = plsc.VectorSubcoreMesh(core_axis_name="c", subcore_axis_name="s")

    @functools.partial(
        pl.kernel,
        out_type=jax.ShapeDtypeStruct((_NC, _L), jnp.float32),
        mesh=mesh,
        scratch_types=[
            pltpu.VMEM((_RPW,), jnp.int32),
            pltpu.VMEM((_NCH, _CH), jnp.int32),
            pltpu.VMEM((_NCH, _CH), jnp.float32),
            pltpu.VMEM((_L,), jnp.float32),
            pltpu.VMEM((_NS * _L,), jnp.float32),
            pltpu.VMEM_SHARED((_NS * _L,), jnp.float32),
            pltpu.SemaphoreType.DMA,
        ],
    )
    def loss_kernel(x_hbm, t_hbm, out_hbm, t_v, idx_v, val_v, acc_v, all_v,
                    part_sh, sem):
        cid = lax.axis_index("c")
        sid = lax.axis_index("s")
        wid = sid * _NC + cid
        base = wid * _RPW

        pltpu.sync_copy(t_hbm.at[pl.ds(base, _RPW)], t_v)

        lanes = lax.iota(jnp.int32, _L)
        for c in range(_NCH):
            for i in range(_CH // _L):
                off = c * _CH + i * _L
                tv = t_v[pl.ds(off, _L)]
                row = base + off + lanes
                idx_v[c, pl.ds(i * _L, _L)] = (
                    ((row >> 3) << 17) + ((tv >> 7) << 10)
                    + ((row & 7) << 7) + (tv & 127)
                )

        copies = [
            pltpu.async_copy(x_hbm.at[idx_v.at[c]], val_v.at[c], sem)
            for c in range(_NCH)
        ]
        for cp in copies:
            cp.wait()

        acc = jnp.zeros((_L,), jnp.float32)
        for c in range(_NCH):
            for i in range(_CH // _L):
                off = c * _CH + i * _L
                tv = t_v[pl.ds(off, _L)]
                v = val_v[c, pl.ds(i * _L, _L)]
                acc = acc - jnp.where(tv != _PAD, v, 0.0)

        acc_v[...] = acc
        pltpu.sync_copy(acc_v, part_sh.at[pl.ds(sid * _L, _L)])
        plsc.subcore_barrier()

        @pl.when(sid == 0)
        def _():
            pltpu.sync_copy(part_sh, all_v)
            tot = jnp.zeros((_L,), jnp.float32)
            for s in range(_NS):
                tot = tot + all_v[pl.ds(s * _L, _L)]
            acc_v[...] = tot
            pltpu.sync_copy(acc_v, out_hbm.at[cid])

    return loss_kernel


_loss_kernel = _make_loss_kernel()


@jax.jit
def kernel(x, target):
    xt = x.reshape(_N // 8, 8, _V // 128, 128)
    xf = xt.transpose(0, 2, 1, 3).reshape(-1)
    t = target.reshape(-1).astype(jnp.int32)
    partials = _loss_kernel(xf, t)
    return jnp.sum(partials)


# speedup vs baseline: 89.6361x; 1.0132x over previous
import functools

import jax
import jax.numpy as jnp
from jax import lax
from jax.experimental import pallas as pl
from jax.experimental.pallas import tpu as pltpu
from jax.experimental.pallas import tpu_sc as plsc

_V = 16384
_N = 4 * 2048
_PAD = 0
_NC = 2
_NS = 16
_NW = _NC * _NS
_RPW = _N // _NW
_CH = 128
_NCH = _RPW // _CH
_L = 16


def _make_loss_kernel():
    mesh = plsc.VectorSubcoreMesh(core_axis_name="c", subcore_axis_name="s")

    @functools.partial(
        pl.kernel,
        out_type=jax.ShapeDtypeStruct((_NW, _L), jnp.float32),
        mesh=mesh,
        scratch_types=[
            pltpu.VMEM((_RPW,), jnp.int32),
            pltpu.VMEM((_RPW,), jnp.int32),
            pltpu.VMEM((_RPW,), jnp.float32),
            pltpu.VMEM((_L,), jnp.float32),
            pltpu.SemaphoreType.DMA,
        ],
    )
    def loss_kernel(x_hbm, t_hbm, out_hbm, t_v, idx_v, val_v, acc_v, sem):
        cid = lax.axis_index("c")
        sid = lax.axis_index("s")
        wid = sid * _NC + cid
        base = wid * _RPW

        pltpu.sync_copy(t_hbm.at[pl.ds(base, _RPW)], t_v)

        lanes = lax.iota(jnp.int32, _L)

        def build(i, _):
            off = i * _L
            tv = t_v[pl.ds(off, _L)]
            row = base + off + lanes
            idx_v[pl.ds(off, _L)] = (
                ((row >> 3) << 17) + ((tv >> 7) << 10)
                + ((row & 7) << 7) + (tv & 127)
            )
            return 0

        lax.fori_loop(0, _RPW // _L, build, 0, unroll=False)

        copies = [
            pltpu.async_copy(x_hbm.at[idx_v.at[pl.ds(c * _CH, _CH)]],
                             val_v.at[pl.ds(c * _CH, _CH)], sem)
            for c in range(_NCH)
        ]
        for cp in copies:
            cp.wait()

        def accum(i, acc):
            off = i * _L
            tv = t_v[pl.ds(off, _L)]
            v = val_v[pl.ds(off, _L)]
            return acc - jnp.where(tv != _PAD, v, 0.0)

        acc = lax.fori_loop(0, _RPW // _L, accum, jnp.zeros((_L,), jnp.float32),
                            unroll=False)

        acc_v[...] = acc
        pltpu.sync_copy(acc_v, out_hbm.at[wid])

    return loss_kernel


_loss_kernel = _make_loss_kernel()


@jax.jit
def kernel(x, target):
    xt = x.reshape(_N // 8, 8, _V // 128, 128)
    xf = xt.transpose(0, 2, 1, 3).reshape(-1)
    t = target.reshape(-1).astype(jnp.int32)
    partials = _loss_kernel(xf, t)
    return jnp.sum(partials)


# speedup vs baseline: 94.4574x; 1.0538x over previous
import functools

import jax
import jax.numpy as jnp
from jax import lax
from jax.experimental import pallas as pl
from jax.experimental.pallas import tpu as pltpu
from jax.experimental.pallas import tpu_sc as plsc

_V = 16384
_N = 4 * 2048
_PAD = 0
_NC = 1
_NS = 16
_NW = _NC * _NS
_RPW = _N // _NW
_CH = 128
_NCH = _RPW // _CH
_L = 16


def _make_loss_kernel():
    mesh = plsc.VectorSubcoreMesh(core_axis_name="c", subcore_axis_name="s",
                                  num_cores=_NC)

    @functools.partial(
        pl.kernel,
        out_type=jax.ShapeDtypeStruct((_NW, _L), jnp.float32),
        mesh=mesh,
        scratch_types=[
            pltpu.VMEM((_RPW,), jnp.int32),
            pltpu.VMEM((_RPW,), jnp.int32),
            pltpu.VMEM((_RPW,), jnp.float32),
            pltpu.VMEM((_L,), jnp.float32),
            pltpu.SemaphoreType.DMA,
        ],
    )
    def loss_kernel(x_hbm, t_hbm, out_hbm, t_v, idx_v, val_v, acc_v, sem):
        cid = lax.axis_index("c")
        sid = lax.axis_index("s")
        wid = sid * _NC + cid
        base = wid * _RPW

        pltpu.sync_copy(t_hbm.at[pl.ds(base, _RPW)], t_v)

        lanes = lax.iota(jnp.int32, _L)

        def build(i, _):
            off = i * _L
            tv = t_v[pl.ds(off, _L)]
            row = base + off + lanes
            idx_v[pl.ds(off, _L)] = (
                ((row >> 3) << 17) + ((tv >> 7) << 10)
                + ((row & 7) << 7) + (tv & 127)
            )
            return 0

        lax.fori_loop(0, _RPW // _L, build, 0, unroll=False)

        copies = [
            pltpu.async_copy(x_hbm.at[idx_v.at[pl.ds(c * _CH, _CH)]],
                             val_v.at[pl.ds(c * _CH, _CH)], sem)
            for c in range(_NCH)
        ]
        for cp in copies:
            cp.wait()

        def accum(i, acc):
            off = i * _L
            tv = t_v[pl.ds(off, _L)]
            v = val_v[pl.ds(off, _L)]
            return acc - jnp.where(tv != _PAD, v, 0.0)

        acc = lax.fori_loop(0, _RPW // _L, accum, jnp.zeros((_L,), jnp.float32),
                            unroll=False)

        acc_v[...] = acc
        pltpu.sync_copy(acc_v, out_hbm.at[wid])

    return loss_kernel


_loss_kernel = _make_loss_kernel()


@jax.jit
def kernel(x, target):
    xt = x.reshape(_N // 8, 8, _V // 128, 128)
    xf = xt.transpose(0, 2, 1, 3).reshape(-1)
    t = target.reshape(-1).astype(jnp.int32)
    partials = _loss_kernel(xf, t)
    return jnp.sum(partials)
